# baseline (speedup 1.0000x reference)
import dataclasses
import functools

import jax
import jax.numpy as jnp
from jax import lax
from jax.experimental import pallas as pl
from jax.experimental.pallas import tpu as pltpu
from jax.experimental.pallas import tpu_sc as plsc

B = 32
N_IN = 3445
N_OUT = 6890
C = 128
SEQ = 9
OUT_C = 128

NW = 32

N_OUT_PAD = 6912
A_ROWS = N_OUT_PAD // NW
A_CHUNK = 128
NNZ_SEARCH = 20736
NNZ_BUF = 20992

G_ROWS = B * N_OUT * SEQ
GB_MACRO = 512
GB_STREAM = 128
N_MACRO = -(-G_ROWS // GB_MACRO)
MACRO_PER_W = -(-N_MACRO // NW)

MM_BLOCK_M = 2048
M_TOTAL = B * N_OUT


def _sc_mesh_params():
    mesh = plsc.VectorSubcoreMesh(core_axis_name="c", subcore_axis_name="s")
    cp = pltpu.CompilerParams()
    if "needs_layout_passes" in pltpu.CompilerParams.__dataclass_fields__:
        cp = dataclasses.replace(cp, needs_layout_passes=False)
    return mesh, cp


def _pooling_sc(x_flat, crv, ks_al, nch):
    mesh, cp = _sc_mesh_params()

    @functools.partial(
        pl.kernel,
        mesh=mesh,
        compiler_params=cp,
        out_type=jax.ShapeDtypeStruct((B * N_OUT_PAD, C), jnp.float32),
        scratch_types=[
            pltpu.VMEM((3, A_CHUNK), jnp.int32),
            pltpu.VMEM((A_CHUNK,), jnp.int32),
            pltpu.VMEM((A_CHUNK,), jnp.int32),
            pltpu.VMEM((A_CHUNK, C), jnp.float32),
            pltpu.VMEM((A_CHUNK, C), jnp.float32),
            pltpu.VMEM((2, A_ROWS, C), jnp.float32),
            pltpu.VMEM((NW,), jnp.int32),
            pltpu.VMEM((NW,), jnp.int32),
            pltpu.SemaphoreType.DMA,
            pltpu.SemaphoreType.DMA,
        ],
    )
    def k(x_hbm, crv_hbm, ks_hbm, nch_hbm, out_hbm,
          crv_v, colva, colvb, xga, xgb, pool2, ks_v, nch_v,
          sema, semb):
        w = lax.axis_index("s") * 2 + lax.axis_index("c")
        pltpu.sync_copy(ks_hbm, ks_v)
        pltpu.sync_copy(nch_hbm, nch_v)
        row_base = w * A_ROWS
        wsplat = jnp.zeros((16,), jnp.int32) + w
        my_ks = plsc.load_gather(ks_v, [wsplat])[0]
        my_nch = plsc.load_gather(nch_v, [wsplat])[0]

        def accumulate(xg, pool):
            @pl.loop(0, A_CHUNK // 16)
            def _(j16):
                base = j16 * 16
                rvec = crv_v[1, pl.ds(base, 16)] - row_base
                vvec = plsc.bitcast(crv_v[2, pl.ds(base, 16)], jnp.float32)
                inrv = jnp.logical_and(rvec >= 0, rvec < A_ROWS)
                vmv = jnp.where(inrv, vvec, jnp.zeros((16,), jnp.float32))
                rcv = jnp.clip(rvec, 0, A_ROWS - 1)
                for jj in range(16):
                    rc = rcv[jj]
                    vb = jnp.zeros((16,), jnp.float32) + vmv[jj]
                    for u in range(C // 16):
                        sl = pl.ds(u * 16, 16)
                        plsc.addupdate(pool.at[rc, sl], xg[base + jj, sl] * vb)

        @pl.loop(0, B // 2)
        def _(bb):
            b0 = bb * 2

            @pl.loop(0, A_ROWS)
            def _(jz):
                for u in range(C // 16):
                    sl = pl.ds(u * 16, 16)
                    pool2[0, jz, sl] = jnp.zeros((16,), jnp.float32)
                    pool2[1, jz, sl] = jnp.zeros((16,), jnp.float32)

            def chunk_body(cc, carry):
                start = pl.multiple_of(my_ks + cc * A_CHUNK, A_CHUNK)
                pltpu.sync_copy(
                    crv_hbm.at[pl.ds(0, 3), pl.ds(start, A_CHUNK)], crv_v)
                for u in range(A_CHUNK // 16):
                    sl = pl.ds(u * 16, 16)
                    ca = crv_v[0, sl] + b0 * N_IN
                    colva[sl] = ca
                    colvb[sl] = ca + N_IN
                ga = pltpu.async_copy(x_hbm.at[colva], xga, sema)
                gb = pltpu.async_copy(x_hbm.at[colvb], xgb, semb)
                ga.wait()
                accumulate(xga, pool2.at[0])
                gb.wait()
                accumulate(xgb, pool2.at[1])
                return carry

            lax.fori_loop(0, my_nch, chunk_body, 0)
            pltpu.sync_copy(
                pool2.at[0],
                out_hbm.at[pl.ds(b0 * N_OUT_PAD + row_base, A_ROWS)])
            pltpu.sync_copy(
                pool2.at[1],
                out_hbm.at[pl.ds((b0 + 1) * N_OUT_PAD + row_base, A_ROWS)])

    return k(x_flat, crv, ks_al, nch)


def _spiral_gather(tab, idx_all):
    mesh, cp = _sc_mesh_params()

    @functools.partial(
        pl.kernel,
        mesh=mesh,
        compiler_params=cp,
        out_type=jax.ShapeDtypeStruct((G_ROWS, C), jnp.float32),
        scratch_types=[
            pltpu.VMEM((GB_MACRO,), jnp.int32),
            pltpu.VMEM((GB_MACRO, C), jnp.float32),
            pltpu.SemaphoreType.DMA,
        ],
    )
    def k(tab_hbm, idx_hbm, out_hbm, idx_v, rows_v, sem):
        w = lax.axis_index("s") * 2 + lax.axis_index("c")

        @pl.loop(0, MACRO_PER_W)
        def _(j):
            mc = w + j * NW

            @pl.when(mc < N_MACRO)
            def _():
                start = jnp.minimum(mc * GB_MACRO, G_ROWS - GB_MACRO)
                pltpu.sync_copy(idx_hbm.at[pl.ds(start, GB_MACRO)], idx_v)
                cps = []
                for t in range(GB_MACRO // GB_STREAM):
                    cps.append(pltpu.async_copy(
                        tab_hbm.at[idx_v.at[pl.ds(t * GB_STREAM, GB_STREAM)]],
                        rows_v.at[pl.ds(t * GB_STREAM, GB_STREAM)],
                        sem,
                    ))
                for cp_ in cps:
                    cp_.wait()
                pltpu.sync_copy(rows_v, out_hbm.at[pl.ds(start, GB_MACRO)])

    return k(tab, idx_all)


def _mm_kernel(sp_ref, w_ref, b_ref, o_ref):
    acc = jnp.dot(sp_ref[0].astype(jnp.bfloat16), w_ref[0],
                  preferred_element_type=jnp.float32)
    for s in range(1, SEQ):
        acc += jnp.dot(sp_ref[s].astype(jnp.bfloat16), w_ref[s],
                       preferred_element_type=jnp.float32)
    o_ref[...] = jnp.maximum(acc + b_ref[...], 0.0)


def _matmul_relu(sp9, wt9, bias):
    grid = (pl.cdiv(M_TOTAL, MM_BLOCK_M),)
    return pl.pallas_call(
        _mm_kernel,
        grid=grid,
        in_specs=[
            pl.BlockSpec((SEQ, MM_BLOCK_M, C), lambda i: (0, i, 0)),
            pl.BlockSpec((SEQ, C, OUT_C), lambda i: (0, 0, 0)),
            pl.BlockSpec((1, OUT_C), lambda i: (0, 0)),
        ],
        out_specs=pl.BlockSpec((MM_BLOCK_M, OUT_C), lambda i: (i, 0)),
        out_shape=jax.ShapeDtypeStruct((M_TOTAL, OUT_C), jnp.float32),
    )(sp9, wt9, bias)


def kernel(x, up_row, up_col, up_val, indices, W, b):
    nnz = up_row.shape[0]
    pad = NNZ_BUF - nnz
    colp = jnp.pad(up_col, (0, pad))
    rowp = jnp.pad(up_row, (0, pad), constant_values=N_OUT)
    valp = jnp.pad(up_val, (0, pad))
    bounds = jnp.searchsorted(
        rowp[:NNZ_SEARCH],
        jnp.arange(NW + 1, dtype=jnp.int32) * A_ROWS).astype(jnp.int32)
    ks = bounds[:NW]
    ke = bounds[1:]
    ks_al = ks - (ks % A_CHUNK)
    nch = (ke - ks_al + A_CHUNK - 1) // A_CHUNK
    crv = jnp.stack([colp, rowp, lax.bitcast_convert_type(valp, jnp.int32)])
    pooled = _pooling_sc(x.reshape(B * N_IN, C), crv, ks_al, nch)
    idx_all = (indices.T[:, None, :]
               + (jnp.arange(B, dtype=jnp.int32) * N_OUT_PAD)[None, :, None])
    sp = _spiral_gather(pooled, idx_all.reshape(-1))
    sp9 = sp.reshape(SEQ, M_TOTAL, C)
    wt9 = W.reshape(OUT_C, SEQ, C).transpose(1, 2, 0).astype(jnp.bfloat16)
    out = _matmul_relu(sp9, wt9, b.reshape(1, OUT_C))
    return out.reshape(B, N_OUT, OUT_C)


# speedup vs baseline: 1.0466x; 1.0466x over previous
import dataclasses
import functools

import jax
import jax.numpy as jnp
from jax import lax
from jax.experimental import pallas as pl
from jax.experimental.pallas import tpu as pltpu
from jax.experimental.pallas import tpu_sc as plsc

B = 32
N_IN = 3445
N_OUT = 6890
C = 128
SEQ = 9
OUT_C = 128

NW = 32

N_OUT_PAD = 6912
A_ROWS = N_OUT_PAD // NW
A_CHUNK = 128
NNZ_SEARCH = 20736
NNZ_BUF = 20992

B_GRP = 8
N_GRP = B // B_GRP

G_ROWS = B_GRP * N_OUT * SEQ
GB_MACRO = 512
GB_STREAM = 128
N_MACRO = -(-G_ROWS // GB_MACRO)
MACRO_PER_W = -(-N_MACRO // NW)

MM_BLOCK_M = 2048
M_TOTAL = B_GRP * N_OUT


def _sc_mesh_params():
    mesh = plsc.VectorSubcoreMesh(core_axis_name="c", subcore_axis_name="s")
    cp = pltpu.CompilerParams()
    if "needs_layout_passes" in pltpu.CompilerParams.__dataclass_fields__:
        cp = dataclasses.replace(cp, needs_layout_passes=False)
    return mesh, cp


def _pooling_sc(x_flat, colp, rowp, valp, ks_al, nch, bstart_nin):
    mesh, cp = _sc_mesh_params()

    @functools.partial(
        pl.kernel,
        mesh=mesh,
        compiler_params=cp,
        out_type=jax.ShapeDtypeStruct((B_GRP * N_OUT_PAD, C), jnp.float32),
        scratch_types=[
            pltpu.VMEM((A_CHUNK,), jnp.int32),
            pltpu.VMEM((A_CHUNK,), jnp.int32),
            pltpu.VMEM((A_CHUNK,), jnp.int32),
            pltpu.VMEM((A_CHUNK, C), jnp.float32),
            pltpu.VMEM((A_CHUNK, C), jnp.float32),
            pltpu.VMEM((2, A_ROWS, C), jnp.float32),
            pltpu.VMEM((A_CHUNK,), jnp.int32),
            pltpu.VMEM((A_CHUNK,), jnp.float32),
            pltpu.VMEM((NW,), jnp.int32),
            pltpu.VMEM((NW,), jnp.int32),
            pltpu.VMEM((8,), jnp.int32),
            pltpu.SemaphoreType.DMA,
            pltpu.SemaphoreType.DMA,
        ],
    )
    def k(x_hbm, col_hbm, row_hbm, val_hbm, ks_hbm, nch_hbm, bs_hbm, out_hbm,
          colv, colva, colvb, xga, xgb, pool2, rowv, valv, ks_v, nch_v,
          bs_v, sema, semb):
        w = lax.axis_index("s") * 2 + lax.axis_index("c")
        pltpu.sync_copy(ks_hbm, ks_v)
        pltpu.sync_copy(nch_hbm, nch_v)
        pltpu.sync_copy(bs_hbm, bs_v)
        row_base = w * A_ROWS
        wsplat = jnp.zeros((16,), jnp.int32) + w
        my_ks = plsc.load_gather(ks_v, [wsplat])[0]
        my_nch = plsc.load_gather(nch_v, [wsplat])[0]
        bs_nin = plsc.load_gather(bs_v, [jnp.zeros((16,), jnp.int32)])[0]

        def accumulate(xg, pool):
            @pl.loop(0, A_CHUNK // 16)
            def _(j16):
                base = j16 * 16
                rvec = rowv[pl.ds(base, 16)] - row_base
                vvec = valv[pl.ds(base, 16)]
                inrv = jnp.logical_and(rvec >= 0, rvec < A_ROWS)
                vmv = jnp.where(inrv, vvec, jnp.zeros((16,), jnp.float32))
                rcv = jnp.clip(rvec, 0, A_ROWS - 1)
                for jj in range(16):
                    rc = rcv[jj]
                    vb = jnp.zeros((16,), jnp.float32) + vmv[jj]
                    for u in range(C // 16):
                        sl = pl.ds(u * 16, 16)
                        plsc.addupdate(pool.at[rc, sl], xg[base + jj, sl] * vb)

        @pl.loop(0, B_GRP // 2)
        def _(bb):
            b0 = bb * 2

            @pl.loop(0, A_ROWS)
            def _(jz):
                for u in range(C // 16):
                    sl = pl.ds(u * 16, 16)
                    pool2[0, jz, sl] = jnp.zeros((16,), jnp.float32)
                    pool2[1, jz, sl] = jnp.zeros((16,), jnp.float32)

            def chunk_body(cc, carry):
                start = pl.multiple_of(my_ks + cc * A_CHUNK, 8)
                pltpu.sync_copy(col_hbm.at[pl.ds(start, A_CHUNK)], colv)
                pltpu.sync_copy(row_hbm.at[pl.ds(start, A_CHUNK)], rowv)
                pltpu.sync_copy(val_hbm.at[pl.ds(start, A_CHUNK)], valv)
                for u in range(A_CHUNK // 16):
                    sl = pl.ds(u * 16, 16)
                    ca = colv[sl] + (bs_nin + b0 * N_IN)
                    colva[sl] = ca
                    colvb[sl] = ca + N_IN
                ga = pltpu.async_copy(x_hbm.at[colva], xga, sema)
                gb = pltpu.async_copy(x_hbm.at[colvb], xgb, semb)
                ga.wait()
                accumulate(xga, pool2.at[0])
                gb.wait()
                accumulate(xgb, pool2.at[1])
                return carry

            lax.fori_loop(0, my_nch, chunk_body, 0)
            pltpu.sync_copy(
                pool2.at[0],
                out_hbm.at[pl.ds(b0 * N_OUT_PAD + row_base, A_ROWS)])
            pltpu.sync_copy(
                pool2.at[1],
                out_hbm.at[pl.ds((b0 + 1) * N_OUT_PAD + row_base, A_ROWS)])

    return k(x_flat, colp, rowp, valp, ks_al, nch, bstart_nin)


def _spiral_gather(tab, idx_all):
    mesh, cp = _sc_mesh_params()

    @functools.partial(
        pl.kernel,
        mesh=mesh,
        compiler_params=cp,
        out_type=jax.ShapeDtypeStruct((G_ROWS, C), jnp.float32),
        scratch_types=[
            pltpu.VMEM((GB_MACRO,), jnp.int32),
            pltpu.VMEM((GB_MACRO, C), jnp.float32),
            pltpu.SemaphoreType.DMA,
        ],
    )
    def k(tab_hbm, idx_hbm, out_hbm, idx_v, rows_v, sem):
        w = lax.axis_index("s") * 2 + lax.axis_index("c")

        @pl.loop(0, MACRO_PER_W)
        def _(j):
            mc = w + j * NW

            @pl.when(mc < N_MACRO)
            def _():
                start = jnp.minimum(mc * GB_MACRO, G_ROWS - GB_MACRO)
                pltpu.sync_copy(idx_hbm.at[pl.ds(start, GB_MACRO)], idx_v)
                cps = []
                for t in range(GB_MACRO // GB_STREAM):
                    cps.append(pltpu.async_copy(
                        tab_hbm.at[idx_v.at[pl.ds(t * GB_STREAM, GB_STREAM)]],
                        rows_v.at[pl.ds(t * GB_STREAM, GB_STREAM)],
                        sem,
                    ))
                for cp_ in cps:
                    cp_.wait()
                pltpu.sync_copy(rows_v, out_hbm.at[pl.ds(start, GB_MACRO)])

    return k(tab, idx_all)


def _mm_kernel(sp_ref, w_ref, b_ref, o_ref):
    acc = jnp.dot(sp_ref[0].astype(jnp.bfloat16), w_ref[0],
                  preferred_element_type=jnp.float32)
    for s in range(1, SEQ):
        acc += jnp.dot(sp_ref[s].astype(jnp.bfloat16), w_ref[s],
                       preferred_element_type=jnp.float32)
    o_ref[...] = jnp.maximum(acc + b_ref[...], 0.0)


def _matmul_relu(sp9, wt9, bias):
    grid = (pl.cdiv(M_TOTAL, MM_BLOCK_M),)
    return pl.pallas_call(
        _mm_kernel,
        grid=grid,
        in_specs=[
            pl.BlockSpec((SEQ, MM_BLOCK_M, C), lambda i: (0, i, 0)),
            pl.BlockSpec((SEQ, C, OUT_C), lambda i: (0, 0, 0)),
            pl.BlockSpec((1, OUT_C), lambda i: (0, 0)),
        ],
        out_specs=pl.BlockSpec((MM_BLOCK_M, OUT_C), lambda i: (i, 0)),
        out_shape=jax.ShapeDtypeStruct((M_TOTAL, OUT_C), jnp.float32),
    )(sp9, wt9, bias)


def kernel(x, up_row, up_col, up_val, indices, W, b):
    nnz = up_row.shape[0]
    pad = NNZ_BUF - nnz
    colp = jnp.pad(up_col, (0, pad))
    rowp = jnp.pad(up_row, (0, pad), constant_values=N_OUT)
    valp = jnp.pad(up_val, (0, pad))
    bounds = jnp.searchsorted(
        rowp[:NNZ_SEARCH],
        jnp.arange(NW + 1, dtype=jnp.int32) * A_ROWS).astype(jnp.int32)
    ks = bounds[:NW]
    ke = bounds[1:]
    ks_al = ks - (ks % 8)
    nch = (ke - ks_al + A_CHUNK - 1) // A_CHUNK
    x_flat = x.reshape(B * N_IN, C)
    idx_grp = (indices.T[:, None, :]
               + (jnp.arange(B_GRP, dtype=jnp.int32) * N_OUT_PAD)[None, :, None]
               ).reshape(-1)
    wt9 = W.reshape(OUT_C, SEQ, C).transpose(1, 2, 0).astype(jnp.bfloat16)
    bias = b.reshape(1, OUT_C)
    outs = []
    for g in range(N_GRP):
        bstart_nin = jnp.full((8,), g * B_GRP * N_IN, jnp.int32)
        pooled = _pooling_sc(x_flat, colp, rowp, valp, ks_al, nch, bstart_nin)
        sp = _spiral_gather(pooled, idx_grp)
        sp9 = sp.reshape(SEQ, M_TOTAL, C)
        out_g = _matmul_relu(sp9, wt9, bias)
        outs.append(out_g.reshape(B_GRP, N_OUT, OUT_C))
    return jnp.concatenate(outs, axis=0)
